# baseline (speedup 1.0000x reference)
import jax
import jax.numpy as jnp
from jax.experimental import pallas as pl
from jax.experimental.pallas import tpu as pltpu

WIDTH = 64
DEPTH = 4
EB = 256
NB_MSG = 16


def _node_prep_body(x_ref, w_ref, b_ref, o_ref):
    o_ref[...] = jnp.dot(x_ref[...], w_ref[...],
                         preferred_element_type=jnp.float32) + b_ref[...]


def _edge_prep_body(ea_ref, w_ref, b_ref, o_ref):
    h = jnp.dot(ea_ref[...], w_ref[...],
                preferred_element_type=jnp.float32) + b_ref[...]
    o_ref[...] = jnp.maximum(h, 0.0).T


def _wgen_body(het_ref, k2_ref, o_ref):
    he = het_ref[...].astype(jnp.bfloat16)
    w = jnp.dot(k2_ref[...], he, preferred_element_type=jnp.float32)
    o_ref[...] = w.astype(jnp.bfloat16)


def _msg_body(hs_ref, wt_ref, b2_ref, h_ref, root_ref, cb_ref, msg_ref, r_ref):
    hs = hs_ref[...]
    hst = hs.T
    acc = jnp.zeros((WIDTH, EB), jnp.float32)
    for i in range(WIDTH):
        w = wt_ref[i * WIDTH:(i + 1) * WIDTH, :].astype(jnp.float32)
        acc = acc + w * hst[i:i + 1, :]
    msg = acc.T + jnp.dot(hs, b2_ref[...], preferred_element_type=jnp.float32)
    msg_ref[...] = msg
    r_ref[...] = jnp.dot(h_ref[...], root_ref[...],
                         preferred_element_type=jnp.float32) + cb_ref[...]


def _update_body(agg_ref, inv_ref, r_ref, o_ref):
    o_ref[...] = jnp.maximum(agg_ref[...] * inv_ref[...] + r_ref[...], 0.0)


def _final_body(agg_ref, inv_ref, r_ref, w2_ref, b2_ref, o_ref):
    h = jnp.maximum(agg_ref[...] * inv_ref[...] + r_ref[...], 0.0)
    o_ref[...] = jnp.sum(h * w2_ref[...], axis=1, keepdims=True) + b2_ref[0, 0]


def kernel(x, edge_index, edge_attr, fc1_W, fc1_b, k1_W, k1_b, k2_W, k2_b,
           root, conv_b, fc2_W, fc2_b):
    N = x.shape[0]
    E = edge_attr.shape[0]
    src = edge_index[0]
    dst = edge_index[1]
    n_eblk = E // EB
    nb = N // n_eblk

    BN = 2000
    h = pl.pallas_call(
        _node_prep_body,
        grid=(N // BN,),
        in_specs=[
            pl.BlockSpec((BN, 3), lambda i: (i, 0)),
            pl.BlockSpec((3, WIDTH), lambda i: (0, 0)),
            pl.BlockSpec((1, WIDTH), lambda i: (0, 0)),
        ],
        out_specs=pl.BlockSpec((BN, WIDTH), lambda i: (i, 0)),
        out_shape=jax.ShapeDtypeStruct((N, WIDTH), jnp.float32),
    )(x, fc1_W, fc1_b.reshape(1, WIDTH))

    het = pl.pallas_call(
        _edge_prep_body,
        grid=(n_eblk,),
        in_specs=[
            pl.BlockSpec((EB, 3), lambda i: (i, 0)),
            pl.BlockSpec((3, WIDTH), lambda i: (0, 0)),
            pl.BlockSpec((1, WIDTH), lambda i: (0, 0)),
        ],
        out_specs=pl.BlockSpec((WIDTH, EB), lambda i: (0, i)),
        out_shape=jax.ShapeDtypeStruct((WIDTH, E), jnp.float32),
    )(edge_attr, k1_W, k1_b.reshape(1, WIDTH))

    k2r2 = k2_W.T.astype(jnp.bfloat16)
    wt = pl.pallas_call(
        _wgen_body,
        grid=(n_eblk,),
        in_specs=[
            pl.BlockSpec((WIDTH, EB), lambda i: (0, i)),
            pl.BlockSpec((WIDTH * WIDTH, WIDTH), lambda i: (0, 0)),
        ],
        out_specs=pl.BlockSpec((WIDTH * WIDTH, EB), lambda i: (0, i)),
        out_shape=jax.ShapeDtypeStruct((WIDTH * WIDTH, E), jnp.bfloat16),
    )(het, k2r2)

    B2 = k2_b.reshape(WIDTH, WIDTH)

    cnt = jax.ops.segment_sum(jnp.ones((E,), jnp.float32), dst, num_segments=N)
    inv_cnt = (1.0 / jnp.clip(cnt, 1.0, None)).reshape(N, 1)

    msg_call = pl.pallas_call(
        _msg_body,
        grid=(n_eblk,),
        in_specs=[
            pl.BlockSpec((EB, WIDTH), lambda i: (i, 0)),
            pl.BlockSpec((WIDTH * WIDTH, EB), lambda i: (0, i)),
            pl.BlockSpec((WIDTH, WIDTH), lambda i: (0, 0)),
            pl.BlockSpec((nb, WIDTH), lambda i: (i, 0)),
            pl.BlockSpec((WIDTH, WIDTH), lambda i: (0, 0)),
            pl.BlockSpec((1, WIDTH), lambda i: (0, 0)),
        ],
        out_specs=[
            pl.BlockSpec((EB, WIDTH), lambda i: (i, 0)),
            pl.BlockSpec((nb, WIDTH), lambda i: (i, 0)),
        ],
        out_shape=[
            jax.ShapeDtypeStruct((E, WIDTH), jnp.float32),
            jax.ShapeDtypeStruct((N, WIDTH), jnp.float32),
        ],
    )

    cbr = conv_b.reshape(1, WIDTH)
    for layer in range(DEPTH):
        hs = h[src]
        msg, r = msg_call(hs, wt, B2, h, root, cbr)
        agg = jax.ops.segment_sum(msg, dst, num_segments=N)
        if layer < DEPTH - 1:
            h = pl.pallas_call(
                _update_body,
                grid=(N // BN,),
                in_specs=[
                    pl.BlockSpec((BN, WIDTH), lambda i: (i, 0)),
                    pl.BlockSpec((BN, 1), lambda i: (i, 0)),
                    pl.BlockSpec((BN, WIDTH), lambda i: (i, 0)),
                ],
                out_specs=pl.BlockSpec((BN, WIDTH), lambda i: (i, 0)),
                out_shape=jax.ShapeDtypeStruct((N, WIDTH), jnp.float32),
            )(agg, inv_cnt, r)
        else:
            out = pl.pallas_call(
                _final_body,
                grid=(N // BN,),
                in_specs=[
                    pl.BlockSpec((BN, WIDTH), lambda i: (i, 0)),
                    pl.BlockSpec((BN, 1), lambda i: (i, 0)),
                    pl.BlockSpec((BN, WIDTH), lambda i: (i, 0)),
                    pl.BlockSpec((1, WIDTH), lambda i: (0, 0)),
                    pl.BlockSpec((1, 1), lambda i: (0, 0)),
                ],
                out_specs=pl.BlockSpec((BN, 1), lambda i: (i, 0)),
                out_shape=jax.ShapeDtypeStruct((N, 1), jnp.float32),
            )(agg, inv_cnt, r, fc2_W.reshape(1, WIDTH), fc2_b.reshape(1, 1))
    return out


# speedup vs baseline: 1.1580x; 1.1580x over previous
import functools

import jax
import jax.numpy as jnp
from jax import lax
from jax.experimental import pallas as pl
from jax.experimental.pallas import tpu as pltpu
from jax.experimental.pallas import tpu_sc as plsc

WIDTH = 64
DEPTH = 4
EB = 256
NC, NS = 2, 16
NW = NC * NS
SCCH = 128
NPAD = 10240
EPAD = 163840
BN = 2048
CW = 16


def _mesh():
    return plsc.VectorSubcoreMesh(core_axis_name="c", subcore_axis_name="s",
                                  num_cores=NC, num_subcores=NS)


def _sc_gather(h_pad, src_pad):
    per_w = EPAD // NW
    nch = per_w // SCCH

    @functools.partial(
        pl.kernel,
        out_type=jax.ShapeDtypeStruct((EPAD, 2 * WIDTH), jnp.float32),
        mesh=_mesh(),
        scratch_types=[
            pltpu.VMEM((SCCH,), jnp.int32),
            pltpu.VMEM((SCCH, 2 * WIDTH), jnp.float32),
            pltpu.SemaphoreType.DMA,
        ],
    )
    def gk(h_ref, src_ref, out_ref, idx_v, rows_v, sem):
        wid = lax.axis_index("s") * NC + lax.axis_index("c")
        base = wid * per_w

        def body(i, carry):
            b = base + i * SCCH
            pltpu.sync_copy(src_ref.at[pl.ds(b, SCCH)], idx_v)
            pltpu.async_copy(h_ref.at[idx_v], rows_v, sem).wait()
            pltpu.sync_copy(rows_v, out_ref.at[pl.ds(b, SCCH)])
            return carry

        lax.fori_loop(0, nch, body, 0)

    return gk(h_pad, src_pad)


def _sc_scatter(msg_pad, dst_pad, zeros128):
    per_w = EPAD // NW
    nch = per_w // SCCH
    rpt = NPAD // NS

    @functools.partial(
        pl.kernel,
        out_type=jax.ShapeDtypeStruct((NC * NPAD, 2 * WIDTH), jnp.float32),
        mesh=_mesh(),
        scratch_types=[
            pltpu.VMEM((SCCH, 2 * WIDTH), jnp.float32),
            pltpu.VMEM((SCCH,), jnp.int32),
            pltpu.VMEM_SHARED((NPAD, 2 * WIDTH), jnp.float32),
        ],
    )
    def sk(msg_ref, dst_ref, z_ref, out_ref, chunk_v, dst_v, agg_sh):
        cid = lax.axis_index("c")
        sid = lax.axis_index("s")
        wid = sid * NC + cid
        pltpu.sync_copy(z_ref.at[pl.ds(sid * rpt, rpt)],
                        agg_sh.at[pl.ds(sid * rpt, rpt)])
        plsc.subcore_barrier()

        def body(i, carry):
            b = wid * per_w + i * SCCH
            pltpu.sync_copy(msg_ref.at[pl.ds(b, SCCH)], chunk_v)
            pltpu.sync_copy(dst_ref.at[pl.ds(b, SCCH)], dst_v)
            pltpu.sync_copy(chunk_v, agg_sh.at[dst_v], add=True)
            return carry

        lax.fori_loop(0, nch, body, 0)
        plsc.subcore_barrier()
        pltpu.sync_copy(agg_sh.at[pl.ds(sid * rpt, rpt)],
                        out_ref.at[pl.ds(cid * NPAD + sid * rpt, rpt)])

    return sk(msg_pad, dst_pad, zeros128)


def _node_prep_body(x_ref, w_ref, b_ref, o_ref):
    h = jnp.dot(x_ref[...], w_ref[...],
                preferred_element_type=jnp.float32) + b_ref[...]
    o_ref[...] = jnp.concatenate([h, jnp.zeros_like(h)], axis=1)


def _edge_prep_body(ea_ref, w_ref, b_ref, o_ref):
    h = jnp.dot(ea_ref[...], w_ref[...],
                preferred_element_type=jnp.float32) + b_ref[...]
    o_ref[...] = jnp.maximum(h, 0.0).T


def _wgen_body(het_ref, k2_ref, o_ref):
    he = het_ref[...].astype(jnp.bfloat16)
    w = jnp.dot(k2_ref[...], he, preferred_element_type=jnp.float32)
    o_ref[...] = w.astype(jnp.bfloat16)


def _msg_body(hs_ref, wt_ref, b2_ref, h_ref, root_ref, cb_ref, msg_ref, r_ref):
    hs = hs_ref[:, :WIDTH]
    hst = hs.T
    acc = jnp.zeros((WIDTH, EB), jnp.float32)
    for i in range(WIDTH):
        w = wt_ref[i * WIDTH:(i + 1) * WIDTH, :].astype(jnp.float32)
        acc = acc + w * hst[i:i + 1, :]
    msg = acc.T + jnp.dot(hs, b2_ref[...], preferred_element_type=jnp.float32)
    col = jax.lax.broadcasted_iota(jnp.int32, (EB, WIDTH), 1)
    ones_col = jnp.where(col == 0, 1.0, 0.0)
    msg_ref[...] = jnp.concatenate([msg, ones_col], axis=1)
    r_ref[...] = jnp.dot(h_ref[:, :WIDTH], root_ref[...],
                         preferred_element_type=jnp.float32) + cb_ref[...]


def _update1_body(a0_ref, a1_ref, r_ref, h_ref, inv_ref):
    c = a0_ref[:, WIDTH:WIDTH + 1] + a1_ref[:, WIDTH:WIDTH + 1]
    inv = 1.0 / jnp.maximum(c, 1.0)
    h = jnp.maximum(
        (a0_ref[:, :WIDTH] + a1_ref[:, :WIDTH]) * inv + r_ref[...], 0.0)
    h_ref[...] = jnp.concatenate([h, jnp.zeros_like(h)], axis=1)
    inv_ref[...] = inv


def _update_body(a0_ref, a1_ref, inv_ref, r_ref, o_ref):
    h = jnp.maximum(
        (a0_ref[:, :WIDTH] + a1_ref[:, :WIDTH]) * inv_ref[...] + r_ref[...],
        0.0)
    o_ref[...] = jnp.concatenate([h, jnp.zeros_like(h)], axis=1)


def _final_body(a0_ref, a1_ref, inv_ref, r_ref, w2_ref, b2_ref, o_ref):
    h = jnp.maximum(
        (a0_ref[:, :WIDTH] + a1_ref[:, :WIDTH]) * inv_ref[...] + r_ref[...],
        0.0)
    o_ref[...] = jnp.sum(h * w2_ref[...], axis=1, keepdims=True) + b2_ref[0, 0]


def _part_specs(bn):
    off = NPAD // bn
    return (pl.BlockSpec((bn, 2 * WIDTH), lambda i: (i, 0)),
            pl.BlockSpec((bn, 2 * WIDTH), lambda i: (i + off, 0)))


def kernel(x, edge_index, edge_attr, fc1_W, fc1_b, k1_W, k1_b, k2_W, k2_b,
           root, conv_b, fc2_W, fc2_b):
    N = x.shape[0]
    E = edge_attr.shape[0]
    src = jnp.pad(edge_index[0], (0, EPAD - E))
    dst = jnp.pad(edge_index[1], (0, EPAD - E), constant_values=NPAD - 1)
    ea_pad = jnp.pad(edge_attr, ((0, EPAD - E), (0, 0)))
    x_pad = jnp.pad(x, ((0, NPAD - N), (0, 0)))
    n_eblk = EPAD // EB
    nb = NPAD // n_eblk

    zeros128 = jnp.zeros((NPAD, 2 * WIDTH), jnp.float32)

    h = pl.pallas_call(
        _node_prep_body,
        grid=(NPAD // BN,),
        in_specs=[
            pl.BlockSpec((BN, 3), lambda i: (i, 0)),
            pl.BlockSpec((3, WIDTH), lambda i: (0, 0)),
            pl.BlockSpec((1, WIDTH), lambda i: (0, 0)),
        ],
        out_specs=pl.BlockSpec((BN, 2 * WIDTH), lambda i: (i, 0)),
        out_shape=jax.ShapeDtypeStruct((NPAD, 2 * WIDTH), jnp.float32),
    )(x_pad, fc1_W, fc1_b.reshape(1, WIDTH))

    het = pl.pallas_call(
        _edge_prep_body,
        grid=(n_eblk,),
        in_specs=[
            pl.BlockSpec((EB, 3), lambda i: (i, 0)),
            pl.BlockSpec((3, WIDTH), lambda i: (0, 0)),
            pl.BlockSpec((1, WIDTH), lambda i: (0, 0)),
        ],
        out_specs=pl.BlockSpec((WIDTH, EB), lambda i: (0, i)),
        out_shape=jax.ShapeDtypeStruct((WIDTH, EPAD), jnp.float32),
    )(ea_pad, k1_W, k1_b.reshape(1, WIDTH))

    k2r2 = k2_W.T.astype(jnp.bfloat16)
    wt = pl.pallas_call(
        _wgen_body,
        grid=(n_eblk,),
        in_specs=[
            pl.BlockSpec((WIDTH, EB), lambda i: (0, i)),
            pl.BlockSpec((WIDTH * WIDTH, WIDTH), lambda i: (0, 0)),
        ],
        out_specs=pl.BlockSpec((WIDTH * WIDTH, EB), lambda i: (0, i)),
        out_shape=jax.ShapeDtypeStruct((WIDTH * WIDTH, EPAD), jnp.bfloat16),
    )(het, k2r2)

    B2 = k2_b.reshape(WIDTH, WIDTH)

    msg_call = pl.pallas_call(
        _msg_body,
        grid=(n_eblk,),
        in_specs=[
            pl.BlockSpec((EB, 2 * WIDTH), lambda i: (i, 0)),
            pl.BlockSpec((WIDTH * WIDTH, EB), lambda i: (0, i)),
            pl.BlockSpec((WIDTH, WIDTH), lambda i: (0, 0)),
            pl.BlockSpec((nb, 2 * WIDTH), lambda i: (i, 0)),
            pl.BlockSpec((WIDTH, WIDTH), lambda i: (0, 0)),
            pl.BlockSpec((1, WIDTH), lambda i: (0, 0)),
        ],
        out_specs=[
            pl.BlockSpec((EB, 2 * WIDTH), lambda i: (i, 0)),
            pl.BlockSpec((nb, WIDTH), lambda i: (i, 0)),
        ],
        out_shape=[
            jax.ShapeDtypeStruct((EPAD, 2 * WIDTH), jnp.float32),
            jax.ShapeDtypeStruct((NPAD, WIDTH), jnp.float32),
        ],
    )

    cbr = conv_b.reshape(1, WIDTH)
    p0_spec, p1_spec = _part_specs(BN)
    coff = NPAD // BN
    inv_cnt = None
    for layer in range(DEPTH):
        hs = _sc_gather(h, src)
        msg, r = msg_call(hs, wt, B2, h, root, cbr)
        parts = _sc_scatter(msg, dst, zeros128)
        if layer == 0:
            h, inv_cnt = pl.pallas_call(
                _update1_body,
                grid=(NPAD // BN,),
                in_specs=[
                    p0_spec, p1_spec,
                    pl.BlockSpec((BN, WIDTH), lambda i: (i, 0)),
                ],
                out_specs=[
                    pl.BlockSpec((BN, 2 * WIDTH), lambda i: (i, 0)),
                    pl.BlockSpec((BN, 1), lambda i: (i, 0)),
                ],
                out_shape=[
                    jax.ShapeDtypeStruct((NPAD, 2 * WIDTH), jnp.float32),
                    jax.ShapeDtypeStruct((NPAD, 1), jnp.float32),
                ],
            )(parts, parts, r)
        elif layer < DEPTH - 1:
            h = pl.pallas_call(
                _update_body,
                grid=(NPAD // BN,),
                in_specs=[
                    p0_spec, p1_spec,
                    pl.BlockSpec((BN, 1), lambda i: (i, 0)),
                    pl.BlockSpec((BN, WIDTH), lambda i: (i, 0)),
                ],
                out_specs=pl.BlockSpec((BN, 2 * WIDTH), lambda i: (i, 0)),
                out_shape=jax.ShapeDtypeStruct((NPAD, 2 * WIDTH), jnp.float32),
            )(parts, parts, inv_cnt, r)
        else:
            out_pad = pl.pallas_call(
                _final_body,
                grid=(NPAD // BN,),
                in_specs=[
                    p0_spec, p1_spec,
                    pl.BlockSpec((BN, 1), lambda i: (i, 0)),
                    pl.BlockSpec((BN, WIDTH), lambda i: (i, 0)),
                    pl.BlockSpec((1, WIDTH), lambda i: (0, 0)),
                    pl.BlockSpec((1, 1), lambda i: (0, 0)),
                ],
                out_specs=pl.BlockSpec((BN, 1), lambda i: (i, 0)),
                out_shape=jax.ShapeDtypeStruct((NPAD, 1), jnp.float32),
            )(parts, parts, inv_cnt, r, fc2_W.reshape(1, WIDTH),
              fc2_b.reshape(1, 1))
            out = out_pad[:N]
    return out


# speedup vs baseline: 1.1993x; 1.0357x over previous
import functools

import jax
import jax.numpy as jnp
from jax import lax
from jax.experimental import pallas as pl
from jax.experimental.pallas import tpu as pltpu
from jax.experimental.pallas import tpu_sc as plsc

WIDTH = 64
DEPTH = 4
EB = 256
NC, NS = 2, 16
NW = NC * NS
SCCH = 128
NPAD = 10240
EPAD = 163840
BN = 2048
CW = 16


def _mesh():
    return plsc.VectorSubcoreMesh(core_axis_name="c", subcore_axis_name="s",
                                  num_cores=NC, num_subcores=NS)


def _sc_gather(h_pad, src_pad):
    per_w = EPAD // NW
    nch = per_w // SCCH

    @functools.partial(
        pl.kernel,
        out_type=jax.ShapeDtypeStruct((EPAD, 2 * WIDTH), jnp.float32),
        mesh=_mesh(),
        scratch_types=[
            pltpu.VMEM((SCCH,), jnp.int32),
            pltpu.VMEM((SCCH,), jnp.int32),
            pltpu.VMEM((SCCH, 2 * WIDTH), jnp.float32),
            pltpu.VMEM((SCCH, 2 * WIDTH), jnp.float32),
            pltpu.SemaphoreType.DMA,
            pltpu.SemaphoreType.DMA,
            pltpu.SemaphoreType.DMA,
            pltpu.SemaphoreType.DMA,
            pltpu.SemaphoreType.DMA,
            pltpu.SemaphoreType.DMA,
        ],
    )
    def gk(h_ref, src_ref, out_ref, idx0, idx1, rows0, rows1,
           si0, si1, sg0, sg1, so0, so1):
        wid = lax.axis_index("s") * NC + lax.axis_index("c")
        base = wid * per_w

        def body(j, carry):
            a = base + (2 * j) * SCCH
            b = base + (2 * j + 1) * SCCH
            ci0 = pltpu.async_copy(src_ref.at[pl.ds(a, SCCH)], idx0, si0)
            ci1 = pltpu.async_copy(src_ref.at[pl.ds(b, SCCH)], idx1, si1)
            ci0.wait()
            g0 = pltpu.async_copy(h_ref.at[idx0], rows0, sg0)
            ci1.wait()
            g1 = pltpu.async_copy(h_ref.at[idx1], rows1, sg1)
            g0.wait()
            o0 = pltpu.async_copy(rows0, out_ref.at[pl.ds(a, SCCH)], so0)
            g1.wait()
            o1 = pltpu.async_copy(rows1, out_ref.at[pl.ds(b, SCCH)], so1)
            o0.wait()
            o1.wait()
            return carry

        lax.fori_loop(0, nch // 2, body, 0)

    return gk(h_pad, src_pad)


def _sc_scatter(msg_pad, dst_pad, zeros128):
    per_w = EPAD // NW
    nch = per_w // SCCH
    rpt = NPAD // NS

    @functools.partial(
        pl.kernel,
        out_type=jax.ShapeDtypeStruct((NC * NPAD, 2 * WIDTH), jnp.float32),
        mesh=_mesh(),
        scratch_types=[
            pltpu.VMEM((SCCH, 2 * WIDTH), jnp.float32),
            pltpu.VMEM((SCCH, 2 * WIDTH), jnp.float32),
            pltpu.VMEM((SCCH,), jnp.int32),
            pltpu.VMEM((SCCH,), jnp.int32),
            pltpu.VMEM_SHARED((NPAD, 2 * WIDTH), jnp.float32),
            pltpu.SemaphoreType.DMA,
            pltpu.SemaphoreType.DMA,
            pltpu.SemaphoreType.DMA,
            pltpu.SemaphoreType.DMA,
            pltpu.SemaphoreType.DMA,
            pltpu.SemaphoreType.DMA,
        ],
    )
    def sk(msg_ref, dst_ref, z_ref, out_ref, chunk0, chunk1, dst0, dst1,
           agg_sh, sm0, sm1, sd0, sd1, sa0, sa1):
        cid = lax.axis_index("c")
        sid = lax.axis_index("s")
        wid = sid * NC + cid
        pltpu.sync_copy(z_ref.at[pl.ds(sid * rpt, rpt)],
                        agg_sh.at[pl.ds(sid * rpt, rpt)])
        plsc.subcore_barrier()

        def body(j, carry):
            a = wid * per_w + (2 * j) * SCCH
            b = wid * per_w + (2 * j + 1) * SCCH
            m0 = pltpu.async_copy(msg_ref.at[pl.ds(a, SCCH)], chunk0, sm0)
            d0 = pltpu.async_copy(dst_ref.at[pl.ds(a, SCCH)], dst0, sd0)
            m1 = pltpu.async_copy(msg_ref.at[pl.ds(b, SCCH)], chunk1, sm1)
            d1 = pltpu.async_copy(dst_ref.at[pl.ds(b, SCCH)], dst1, sd1)
            m0.wait()
            d0.wait()
            a0 = pltpu.async_copy(chunk0, agg_sh.at[dst0], sa0, add=True)
            m1.wait()
            d1.wait()
            a1 = pltpu.async_copy(chunk1, agg_sh.at[dst1], sa1, add=True)
            a0.wait()
            a1.wait()
            return carry

        lax.fori_loop(0, nch // 2, body, 0)
        plsc.subcore_barrier()
        pltpu.sync_copy(agg_sh.at[pl.ds(sid * rpt, rpt)],
                        out_ref.at[pl.ds(cid * NPAD + sid * rpt, rpt)])

    return sk(msg_pad, dst_pad, zeros128)


def _node_prep_body(x_ref, w_ref, b_ref, o_ref):
    h = jnp.dot(x_ref[...], w_ref[...],
                preferred_element_type=jnp.float32) + b_ref[...]
    o_ref[...] = jnp.concatenate([h, jnp.zeros_like(h)], axis=1)


def _edge_prep_body(ea_ref, w_ref, b_ref, o_ref):
    h = jnp.dot(ea_ref[...], w_ref[...],
                preferred_element_type=jnp.float32) + b_ref[...]
    o_ref[...] = jnp.maximum(h, 0.0).T


def _wgen_body(het_ref, k2_ref, o_ref):
    he = het_ref[...].astype(jnp.bfloat16)
    w = jnp.dot(k2_ref[...], he, preferred_element_type=jnp.float32)
    o_ref[...] = w.astype(jnp.bfloat16)


def _msg_body(hs_ref, wt_ref, b2_ref, h_ref, root_ref, cb_ref, msg_ref, r_ref):
    hs = hs_ref[:, :WIDTH]
    hst = hs.T
    acc = jnp.zeros((WIDTH, EB), jnp.float32)
    for i in range(WIDTH):
        w = wt_ref[i * WIDTH:(i + 1) * WIDTH, :].astype(jnp.float32)
        acc = acc + w * hst[i:i + 1, :]
    msg = acc.T + jnp.dot(hs, b2_ref[...], preferred_element_type=jnp.float32)
    col = jax.lax.broadcasted_iota(jnp.int32, (EB, WIDTH), 1)
    ones_col = jnp.where(col == 0, 1.0, 0.0)
    msg_ref[...] = jnp.concatenate([msg, ones_col], axis=1)
    r_ref[...] = jnp.dot(h_ref[:, :WIDTH], root_ref[...],
                         preferred_element_type=jnp.float32) + cb_ref[...]


def _update1_body(a0_ref, a1_ref, r_ref, h_ref, inv_ref):
    c = a0_ref[:, WIDTH:WIDTH + 1] + a1_ref[:, WIDTH:WIDTH + 1]
    inv = 1.0 / jnp.maximum(c, 1.0)
    h = jnp.maximum(
        (a0_ref[:, :WIDTH] + a1_ref[:, :WIDTH]) * inv + r_ref[...], 0.0)
    h_ref[...] = jnp.concatenate([h, jnp.zeros_like(h)], axis=1)
    inv_ref[...] = inv


def _update_body(a0_ref, a1_ref, inv_ref, r_ref, o_ref):
    h = jnp.maximum(
        (a0_ref[:, :WIDTH] + a1_ref[:, :WIDTH]) * inv_ref[...] + r_ref[...],
        0.0)
    o_ref[...] = jnp.concatenate([h, jnp.zeros_like(h)], axis=1)


def _final_body(a0_ref, a1_ref, inv_ref, r_ref, w2_ref, b2_ref, o_ref):
    h = jnp.maximum(
        (a0_ref[:, :WIDTH] + a1_ref[:, :WIDTH]) * inv_ref[...] + r_ref[...],
        0.0)
    o_ref[...] = jnp.sum(h * w2_ref[...], axis=1, keepdims=True) + b2_ref[0, 0]


def _part_specs(bn):
    off = NPAD // bn
    return (pl.BlockSpec((bn, 2 * WIDTH), lambda i: (i, 0)),
            pl.BlockSpec((bn, 2 * WIDTH), lambda i: (i + off, 0)))


def kernel(x, edge_index, edge_attr, fc1_W, fc1_b, k1_W, k1_b, k2_W, k2_b,
           root, conv_b, fc2_W, fc2_b):
    N = x.shape[0]
    E = edge_attr.shape[0]
    src = jnp.pad(edge_index[0], (0, EPAD - E))
    dst = jnp.pad(edge_index[1], (0, EPAD - E), constant_values=NPAD - 1)
    ea_pad = jnp.pad(edge_attr, ((0, EPAD - E), (0, 0)))
    x_pad = jnp.pad(x, ((0, NPAD - N), (0, 0)))
    n_eblk = EPAD // EB
    nb = NPAD // n_eblk

    zeros128 = jnp.zeros((NPAD, 2 * WIDTH), jnp.float32)

    h = pl.pallas_call(
        _node_prep_body,
        grid=(NPAD // BN,),
        in_specs=[
            pl.BlockSpec((BN, 3), lambda i: (i, 0)),
            pl.BlockSpec((3, WIDTH), lambda i: (0, 0)),
            pl.BlockSpec((1, WIDTH), lambda i: (0, 0)),
        ],
        out_specs=pl.BlockSpec((BN, 2 * WIDTH), lambda i: (i, 0)),
        out_shape=jax.ShapeDtypeStruct((NPAD, 2 * WIDTH), jnp.float32),
    )(x_pad, fc1_W, fc1_b.reshape(1, WIDTH))

    het = pl.pallas_call(
        _edge_prep_body,
        grid=(n_eblk,),
        in_specs=[
            pl.BlockSpec((EB, 3), lambda i: (i, 0)),
            pl.BlockSpec((3, WIDTH), lambda i: (0, 0)),
            pl.BlockSpec((1, WIDTH), lambda i: (0, 0)),
        ],
        out_specs=pl.BlockSpec((WIDTH, EB), lambda i: (0, i)),
        out_shape=jax.ShapeDtypeStruct((WIDTH, EPAD), jnp.float32),
    )(ea_pad, k1_W, k1_b.reshape(1, WIDTH))

    k2r2 = k2_W.T.astype(jnp.bfloat16)
    wt = pl.pallas_call(
        _wgen_body,
        grid=(n_eblk,),
        in_specs=[
            pl.BlockSpec((WIDTH, EB), lambda i: (0, i)),
            pl.BlockSpec((WIDTH * WIDTH, WIDTH), lambda i: (0, 0)),
        ],
        out_specs=pl.BlockSpec((WIDTH * WIDTH, EB), lambda i: (0, i)),
        out_shape=jax.ShapeDtypeStruct((WIDTH * WIDTH, EPAD), jnp.bfloat16),
    )(het, k2r2)

    B2 = k2_b.reshape(WIDTH, WIDTH)

    msg_call = pl.pallas_call(
        _msg_body,
        grid=(n_eblk,),
        in_specs=[
            pl.BlockSpec((EB, 2 * WIDTH), lambda i: (i, 0)),
            pl.BlockSpec((WIDTH * WIDTH, EB), lambda i: (0, i)),
            pl.BlockSpec((WIDTH, WIDTH), lambda i: (0, 0)),
            pl.BlockSpec((nb, 2 * WIDTH), lambda i: (i, 0)),
            pl.BlockSpec((WIDTH, WIDTH), lambda i: (0, 0)),
            pl.BlockSpec((1, WIDTH), lambda i: (0, 0)),
        ],
        out_specs=[
            pl.BlockSpec((EB, 2 * WIDTH), lambda i: (i, 0)),
            pl.BlockSpec((nb, WIDTH), lambda i: (i, 0)),
        ],
        out_shape=[
            jax.ShapeDtypeStruct((EPAD, 2 * WIDTH), jnp.float32),
            jax.ShapeDtypeStruct((NPAD, WIDTH), jnp.float32),
        ],
    )

    cbr = conv_b.reshape(1, WIDTH)
    p0_spec, p1_spec = _part_specs(BN)
    coff = NPAD // BN
    inv_cnt = None
    for layer in range(DEPTH):
        hs = _sc_gather(h, src)
        msg, r = msg_call(hs, wt, B2, h, root, cbr)
        parts = _sc_scatter(msg, dst, zeros128)
        if layer == 0:
            h, inv_cnt = pl.pallas_call(
                _update1_body,
                grid=(NPAD // BN,),
                in_specs=[
                    p0_spec, p1_spec,
                    pl.BlockSpec((BN, WIDTH), lambda i: (i, 0)),
                ],
                out_specs=[
                    pl.BlockSpec((BN, 2 * WIDTH), lambda i: (i, 0)),
                    pl.BlockSpec((BN, 1), lambda i: (i, 0)),
                ],
                out_shape=[
                    jax.ShapeDtypeStruct((NPAD, 2 * WIDTH), jnp.float32),
                    jax.ShapeDtypeStruct((NPAD, 1), jnp.float32),
                ],
            )(parts, parts, r)
        elif layer < DEPTH - 1:
            h = pl.pallas_call(
                _update_body,
                grid=(NPAD // BN,),
                in_specs=[
                    p0_spec, p1_spec,
                    pl.BlockSpec((BN, 1), lambda i: (i, 0)),
                    pl.BlockSpec((BN, WIDTH), lambda i: (i, 0)),
                ],
                out_specs=pl.BlockSpec((BN, 2 * WIDTH), lambda i: (i, 0)),
                out_shape=jax.ShapeDtypeStruct((NPAD, 2 * WIDTH), jnp.float32),
            )(parts, parts, inv_cnt, r)
        else:
            out_pad = pl.pallas_call(
                _final_body,
                grid=(NPAD // BN,),
                in_specs=[
                    p0_spec, p1_spec,
                    pl.BlockSpec((BN, 1), lambda i: (i, 0)),
                    pl.BlockSpec((BN, WIDTH), lambda i: (i, 0)),
                    pl.BlockSpec((1, WIDTH), lambda i: (0, 0)),
                    pl.BlockSpec((1, 1), lambda i: (0, 0)),
                ],
                out_specs=pl.BlockSpec((BN, 1), lambda i: (i, 0)),
                out_shape=jax.ShapeDtypeStruct((NPAD, 1), jnp.float32),
            )(parts, parts, inv_cnt, r, fc2_W.reshape(1, WIDTH),
              fc2_b.reshape(1, 1))
            out = out_pad[:N]
    return out


# speedup vs baseline: 1.4896x; 1.2421x over previous
import functools

import jax
import jax.numpy as jnp
from jax import lax
from jax.experimental import pallas as pl
from jax.experimental.pallas import tpu as pltpu
from jax.experimental.pallas import tpu_sc as plsc

WIDTH = 64
DEPTH = 4
EB = 512
NC, NS = 2, 16
NW = NC * NS
SCCH = 128
NPAD = 10240
EPAD = 163840
BN = 2048
CW = 16


def _mesh():
    return plsc.VectorSubcoreMesh(core_axis_name="c", subcore_axis_name="s",
                                  num_cores=NC, num_subcores=NS)


def _sc_gather(h_pad, src_pad):
    per_w = EPAD // NW
    nch = per_w // SCCH

    @functools.partial(
        pl.kernel,
        out_type=jax.ShapeDtypeStruct((EPAD, 2 * WIDTH), jnp.float32),
        mesh=_mesh(),
        scratch_types=[
            pltpu.VMEM((SCCH,), jnp.int32),
            pltpu.VMEM((SCCH,), jnp.int32),
            pltpu.VMEM((SCCH, 2 * WIDTH), jnp.float32),
            pltpu.VMEM((SCCH, 2 * WIDTH), jnp.float32),
            pltpu.SemaphoreType.DMA,
            pltpu.SemaphoreType.DMA,
            pltpu.SemaphoreType.DMA,
            pltpu.SemaphoreType.DMA,
            pltpu.SemaphoreType.DMA,
            pltpu.SemaphoreType.DMA,
        ],
    )
    def gk(h_ref, src_ref, out_ref, idx0, idx1, rows0, rows1,
           si0, si1, sg0, sg1, so0, so1):
        wid = lax.axis_index("s") * NC + lax.axis_index("c")
        base = wid * per_w

        def body(j, carry):
            a = base + (2 * j) * SCCH
            b = base + (2 * j + 1) * SCCH
            ci0 = pltpu.async_copy(src_ref.at[pl.ds(a, SCCH)], idx0, si0)
            ci1 = pltpu.async_copy(src_ref.at[pl.ds(b, SCCH)], idx1, si1)
            ci0.wait()
            g0 = pltpu.async_copy(h_ref.at[idx0], rows0, sg0)
            ci1.wait()
            g1 = pltpu.async_copy(h_ref.at[idx1], rows1, sg1)
            g0.wait()
            o0 = pltpu.async_copy(rows0, out_ref.at[pl.ds(a, SCCH)], so0)
            g1.wait()
            o1 = pltpu.async_copy(rows1, out_ref.at[pl.ds(b, SCCH)], so1)
            o0.wait()
            o1.wait()
            return carry

        lax.fori_loop(0, nch // 2, body, 0)

    return gk(h_pad, src_pad)


def _sc_scatter(msg_pad, dst_pad, zeros128):
    per_w = EPAD // NW
    nch = per_w // SCCH
    rpt = NPAD // NS

    @functools.partial(
        pl.kernel,
        out_type=jax.ShapeDtypeStruct((NC * NPAD, 2 * WIDTH), jnp.float32),
        mesh=_mesh(),
        scratch_types=[
            pltpu.VMEM((SCCH, 2 * WIDTH), jnp.float32),
            pltpu.VMEM((SCCH, 2 * WIDTH), jnp.float32),
            pltpu.VMEM((SCCH,), jnp.int32),
            pltpu.VMEM((SCCH,), jnp.int32),
            pltpu.VMEM_SHARED((NPAD, 2 * WIDTH), jnp.float32),
            pltpu.SemaphoreType.DMA,
            pltpu.SemaphoreType.DMA,
            pltpu.SemaphoreType.DMA,
            pltpu.SemaphoreType.DMA,
            pltpu.SemaphoreType.DMA,
            pltpu.SemaphoreType.DMA,
        ],
    )
    def sk(msg_ref, dst_ref, z_ref, out_ref, chunk0, chunk1, dst0, dst1,
           agg_sh, sm0, sm1, sd0, sd1, sa0, sa1):
        cid = lax.axis_index("c")
        sid = lax.axis_index("s")
        wid = sid * NC + cid
        pltpu.sync_copy(z_ref.at[pl.ds(sid * rpt, rpt)],
                        agg_sh.at[pl.ds(sid * rpt, rpt)])
        plsc.subcore_barrier()

        def body(j, carry):
            a = wid * per_w + (2 * j) * SCCH
            b = wid * per_w + (2 * j + 1) * SCCH
            m0 = pltpu.async_copy(msg_ref.at[pl.ds(a, SCCH)], chunk0, sm0)
            d0 = pltpu.async_copy(dst_ref.at[pl.ds(a, SCCH)], dst0, sd0)
            m1 = pltpu.async_copy(msg_ref.at[pl.ds(b, SCCH)], chunk1, sm1)
            d1 = pltpu.async_copy(dst_ref.at[pl.ds(b, SCCH)], dst1, sd1)
            m0.wait()
            d0.wait()
            a0 = pltpu.async_copy(chunk0, agg_sh.at[dst0], sa0, add=True)
            m1.wait()
            d1.wait()
            a1 = pltpu.async_copy(chunk1, agg_sh.at[dst1], sa1, add=True)
            a0.wait()
            a1.wait()
            return carry

        lax.fori_loop(0, nch // 2, body, 0)
        plsc.subcore_barrier()
        pltpu.sync_copy(agg_sh.at[pl.ds(sid * rpt, rpt)],
                        out_ref.at[pl.ds(cid * NPAD + sid * rpt, rpt)])

    return sk(msg_pad, dst_pad, zeros128)


def _node_prep_body(x_ref, w_ref, b_ref, o_ref):
    h = jnp.dot(x_ref[...], w_ref[...],
                preferred_element_type=jnp.float32) + b_ref[...]
    o_ref[...] = jnp.concatenate([h, jnp.zeros_like(h)], axis=1)


def _edge_prep_body(ea_ref, w_ref, b_ref, o_ref):
    h = jnp.dot(ea_ref[...], w_ref[...],
                preferred_element_type=jnp.float32) + b_ref[...]
    o_ref[...] = jnp.maximum(h, 0.0).T


def _wgen_body(het_ref, k2_ref, o_ref):
    he = het_ref[...].astype(jnp.bfloat16)
    w = jnp.dot(k2_ref[...], he, preferred_element_type=jnp.float32)
    o_ref[...] = w.astype(jnp.bfloat16)


def _msg_body(hs_ref, wt_ref, b2_ref, h_ref, root_ref, cb_ref, msg_ref, r_ref):
    hs = hs_ref[:, :WIDTH]
    hst = hs.T
    acc = jnp.zeros((WIDTH, EB), jnp.float32)
    for i in range(WIDTH):
        w = wt_ref[i * WIDTH:(i + 1) * WIDTH, :].astype(jnp.float32)
        acc = acc + w * hst[i:i + 1, :]
    msg = acc.T + jnp.dot(hs, b2_ref[...], preferred_element_type=jnp.float32)
    col = jax.lax.broadcasted_iota(jnp.int32, (EB, WIDTH), 1)
    ones_col = jnp.where(col == 0, 1.0, 0.0)
    msg_ref[...] = jnp.concatenate([msg, ones_col], axis=1)
    r_ref[...] = jnp.dot(h_ref[:, :WIDTH], root_ref[...],
                         preferred_element_type=jnp.float32) + cb_ref[...]


def _update1_body(a0_ref, a1_ref, r_ref, h_ref, inv_ref):
    c = a0_ref[:, WIDTH:WIDTH + 1] + a1_ref[:, WIDTH:WIDTH + 1]
    inv = 1.0 / jnp.maximum(c, 1.0)
    h = jnp.maximum(
        (a0_ref[:, :WIDTH] + a1_ref[:, :WIDTH]) * inv + r_ref[...], 0.0)
    h_ref[...] = jnp.concatenate([h, jnp.zeros_like(h)], axis=1)
    inv_ref[...] = inv


def _update_body(a0_ref, a1_ref, inv_ref, r_ref, o_ref):
    h = jnp.maximum(
        (a0_ref[:, :WIDTH] + a1_ref[:, :WIDTH]) * inv_ref[...] + r_ref[...],
        0.0)
    o_ref[...] = jnp.concatenate([h, jnp.zeros_like(h)], axis=1)


def _final_body(a0_ref, a1_ref, inv_ref, r_ref, w2_ref, b2_ref, o_ref):
    h = jnp.maximum(
        (a0_ref[:, :WIDTH] + a1_ref[:, :WIDTH]) * inv_ref[...] + r_ref[...],
        0.0)
    o_ref[...] = jnp.sum(h * w2_ref[...], axis=1, keepdims=True) + b2_ref[0, 0]


def _part_specs(bn):
    off = NPAD // bn
    return (pl.BlockSpec((bn, 2 * WIDTH), lambda i: (i, 0)),
            pl.BlockSpec((bn, 2 * WIDTH), lambda i: (i + off, 0)))


def kernel(x, edge_index, edge_attr, fc1_W, fc1_b, k1_W, k1_b, k2_W, k2_b,
           root, conv_b, fc2_W, fc2_b):
    N = x.shape[0]
    E = edge_attr.shape[0]
    src = jnp.pad(edge_index[0], (0, EPAD - E))
    dst = jnp.pad(edge_index[1], (0, EPAD - E), constant_values=NPAD - 1)
    ea_pad = jnp.pad(edge_attr, ((0, EPAD - E), (0, 0)))
    x_pad = jnp.pad(x, ((0, NPAD - N), (0, 0)))
    n_eblk = EPAD // EB
    nb = NPAD // n_eblk

    zeros128 = jnp.zeros((NPAD, 2 * WIDTH), jnp.float32)

    h = pl.pallas_call(
        _node_prep_body,
        grid=(NPAD // BN,),
        in_specs=[
            pl.BlockSpec((BN, 3), lambda i: (i, 0)),
            pl.BlockSpec((3, WIDTH), lambda i: (0, 0)),
            pl.BlockSpec((1, WIDTH), lambda i: (0, 0)),
        ],
        out_specs=pl.BlockSpec((BN, 2 * WIDTH), lambda i: (i, 0)),
        out_shape=jax.ShapeDtypeStruct((NPAD, 2 * WIDTH), jnp.float32),
    )(x_pad, fc1_W, fc1_b.reshape(1, WIDTH))

    het = pl.pallas_call(
        _edge_prep_body,
        grid=(n_eblk,),
        in_specs=[
            pl.BlockSpec((EB, 3), lambda i: (i, 0)),
            pl.BlockSpec((3, WIDTH), lambda i: (0, 0)),
            pl.BlockSpec((1, WIDTH), lambda i: (0, 0)),
        ],
        out_specs=pl.BlockSpec((WIDTH, EB), lambda i: (0, i)),
        out_shape=jax.ShapeDtypeStruct((WIDTH, EPAD), jnp.float32),
    )(ea_pad, k1_W, k1_b.reshape(1, WIDTH))

    k2r2 = k2_W.T.astype(jnp.bfloat16)
    wt = pl.pallas_call(
        _wgen_body,
        grid=(n_eblk,),
        in_specs=[
            pl.BlockSpec((WIDTH, EB), lambda i: (0, i)),
            pl.BlockSpec((WIDTH * WIDTH, WIDTH), lambda i: (0, 0)),
        ],
        out_specs=pl.BlockSpec((WIDTH * WIDTH, EB), lambda i: (0, i)),
        out_shape=jax.ShapeDtypeStruct((WIDTH * WIDTH, EPAD), jnp.bfloat16),
    )(het, k2r2)

    B2 = k2_b.reshape(WIDTH, WIDTH)

    msg_call = pl.pallas_call(
        _msg_body,
        grid=(n_eblk,),
        in_specs=[
            pl.BlockSpec((EB, 2 * WIDTH), lambda i: (i, 0)),
            pl.BlockSpec((WIDTH * WIDTH, EB), lambda i: (0, i)),
            pl.BlockSpec((WIDTH, WIDTH), lambda i: (0, 0)),
            pl.BlockSpec((nb, 2 * WIDTH), lambda i: (i, 0)),
            pl.BlockSpec((WIDTH, WIDTH), lambda i: (0, 0)),
            pl.BlockSpec((1, WIDTH), lambda i: (0, 0)),
        ],
        out_specs=[
            pl.BlockSpec((EB, 2 * WIDTH), lambda i: (i, 0)),
            pl.BlockSpec((nb, WIDTH), lambda i: (i, 0)),
        ],
        out_shape=[
            jax.ShapeDtypeStruct((EPAD, 2 * WIDTH), jnp.float32),
            jax.ShapeDtypeStruct((NPAD, WIDTH), jnp.float32),
        ],
    )

    cbr = conv_b.reshape(1, WIDTH)
    p0_spec, p1_spec = _part_specs(BN)
    coff = NPAD // BN
    inv_cnt = None
    for layer in range(DEPTH):
        hs = _sc_gather(h, src)
        msg, r = msg_call(hs, wt, B2, h, root, cbr)
        parts = _sc_scatter(msg, dst, zeros128)
        if layer == 0:
            h, inv_cnt = pl.pallas_call(
                _update1_body,
                grid=(NPAD // BN,),
                in_specs=[
                    p0_spec, p1_spec,
                    pl.BlockSpec((BN, WIDTH), lambda i: (i, 0)),
                ],
                out_specs=[
                    pl.BlockSpec((BN, 2 * WIDTH), lambda i: (i, 0)),
                    pl.BlockSpec((BN, 1), lambda i: (i, 0)),
                ],
                out_shape=[
                    jax.ShapeDtypeStruct((NPAD, 2 * WIDTH), jnp.float32),
                    jax.ShapeDtypeStruct((NPAD, 1), jnp.float32),
                ],
            )(parts, parts, r)
        elif layer < DEPTH - 1:
            h = pl.pallas_call(
                _update_body,
                grid=(NPAD // BN,),
                in_specs=[
                    p0_spec, p1_spec,
                    pl.BlockSpec((BN, 1), lambda i: (i, 0)),
                    pl.BlockSpec((BN, WIDTH), lambda i: (i, 0)),
                ],
                out_specs=pl.BlockSpec((BN, 2 * WIDTH), lambda i: (i, 0)),
                out_shape=jax.ShapeDtypeStruct((NPAD, 2 * WIDTH), jnp.float32),
            )(parts, parts, inv_cnt, r)
        else:
            out_pad = pl.pallas_call(
                _final_body,
                grid=(NPAD // BN,),
                in_specs=[
                    p0_spec, p1_spec,
                    pl.BlockSpec((BN, 1), lambda i: (i, 0)),
                    pl.BlockSpec((BN, WIDTH), lambda i: (i, 0)),
                    pl.BlockSpec((1, WIDTH), lambda i: (0, 0)),
                    pl.BlockSpec((1, 1), lambda i: (0, 0)),
                ],
                out_specs=pl.BlockSpec((BN, 1), lambda i: (i, 0)),
                out_shape=jax.ShapeDtypeStruct((NPAD, 1), jnp.float32),
            )(parts, parts, inv_cnt, r, fc2_W.reshape(1, WIDTH),
              fc2_b.reshape(1, 1))
            out = out_pad[:N]
    return out


# speedup vs baseline: 1.6729x; 1.1231x over previous
import functools

import jax
import jax.numpy as jnp
from jax import lax
from jax.experimental import pallas as pl
from jax.experimental.pallas import tpu as pltpu
from jax.experimental.pallas import tpu_sc as plsc

WIDTH = 64
DEPTH = 4
EB = 1024
NC, NS = 2, 16
NW = NC * NS
SCCH = 128
NPAD = 10240
EPAD = 163840
BN = 2048
CW = 16


def _mesh():
    return plsc.VectorSubcoreMesh(core_axis_name="c", subcore_axis_name="s",
                                  num_cores=NC, num_subcores=NS)


def _sc_gather(h_pad, src_pad):
    per_w = EPAD // NW
    nch = per_w // SCCH

    @functools.partial(
        pl.kernel,
        out_type=jax.ShapeDtypeStruct((EPAD, 2 * WIDTH), jnp.float32),
        mesh=_mesh(),
        scratch_types=[
            pltpu.VMEM((SCCH,), jnp.int32),
            pltpu.VMEM((SCCH,), jnp.int32),
            pltpu.VMEM((SCCH, 2 * WIDTH), jnp.float32),
            pltpu.VMEM((SCCH, 2 * WIDTH), jnp.float32),
            pltpu.SemaphoreType.DMA,
            pltpu.SemaphoreType.DMA,
            pltpu.SemaphoreType.DMA,
            pltpu.SemaphoreType.DMA,
            pltpu.SemaphoreType.DMA,
            pltpu.SemaphoreType.DMA,
        ],
    )
    def gk(h_ref, src_ref, out_ref, idx0, idx1, rows0, rows1,
           si0, si1, sg0, sg1, so0, so1):
        wid = lax.axis_index("s") * NC + lax.axis_index("c")
        base = wid * per_w

        def body(j, carry):
            a = base + (2 * j) * SCCH
            b = base + (2 * j + 1) * SCCH
            ci0 = pltpu.async_copy(src_ref.at[pl.ds(a, SCCH)], idx0, si0)
            ci1 = pltpu.async_copy(src_ref.at[pl.ds(b, SCCH)], idx1, si1)
            ci0.wait()
            g0 = pltpu.async_copy(h_ref.at[idx0], rows0, sg0)
            ci1.wait()
            g1 = pltpu.async_copy(h_ref.at[idx1], rows1, sg1)
            g0.wait()
            o0 = pltpu.async_copy(rows0, out_ref.at[pl.ds(a, SCCH)], so0)
            g1.wait()
            o1 = pltpu.async_copy(rows1, out_ref.at[pl.ds(b, SCCH)], so1)
            o0.wait()
            o1.wait()
            return carry

        lax.fori_loop(0, nch // 2, body, 0)

    return gk(h_pad, src_pad)


def _sc_scatter(msg_pad, dst_pad, zeros128):
    per_w = EPAD // NW
    nch = per_w // SCCH
    rpt = NPAD // NS

    @functools.partial(
        pl.kernel,
        out_type=jax.ShapeDtypeStruct((NC * NPAD, 2 * WIDTH), jnp.float32),
        mesh=_mesh(),
        scratch_types=[
            pltpu.VMEM((SCCH, 2 * WIDTH), jnp.float32),
            pltpu.VMEM((SCCH, 2 * WIDTH), jnp.float32),
            pltpu.VMEM((SCCH,), jnp.int32),
            pltpu.VMEM((SCCH,), jnp.int32),
            pltpu.VMEM_SHARED((NPAD, 2 * WIDTH), jnp.float32),
            pltpu.SemaphoreType.DMA,
            pltpu.SemaphoreType.DMA,
            pltpu.SemaphoreType.DMA,
            pltpu.SemaphoreType.DMA,
            pltpu.SemaphoreType.DMA,
            pltpu.SemaphoreType.DMA,
        ],
    )
    def sk(msg_ref, dst_ref, z_ref, out_ref, chunk0, chunk1, dst0, dst1,
           agg_sh, sm0, sm1, sd0, sd1, sa0, sa1):
        cid = lax.axis_index("c")
        sid = lax.axis_index("s")
        wid = sid * NC + cid
        pltpu.sync_copy(z_ref.at[pl.ds(sid * rpt, rpt)],
                        agg_sh.at[pl.ds(sid * rpt, rpt)])
        plsc.subcore_barrier()

        def body(j, carry):
            a = wid * per_w + (2 * j) * SCCH
            b = wid * per_w + (2 * j + 1) * SCCH
            m0 = pltpu.async_copy(msg_ref.at[pl.ds(a, SCCH)], chunk0, sm0)
            d0 = pltpu.async_copy(dst_ref.at[pl.ds(a, SCCH)], dst0, sd0)
            m1 = pltpu.async_copy(msg_ref.at[pl.ds(b, SCCH)], chunk1, sm1)
            d1 = pltpu.async_copy(dst_ref.at[pl.ds(b, SCCH)], dst1, sd1)
            m0.wait()
            d0.wait()
            a0 = pltpu.async_copy(chunk0, agg_sh.at[dst0], sa0, add=True)
            m1.wait()
            d1.wait()
            a1 = pltpu.async_copy(chunk1, agg_sh.at[dst1], sa1, add=True)
            a0.wait()
            a1.wait()
            return carry

        lax.fori_loop(0, nch // 2, body, 0)
        plsc.subcore_barrier()
        pltpu.sync_copy(agg_sh.at[pl.ds(sid * rpt, rpt)],
                        out_ref.at[pl.ds(cid * NPAD + sid * rpt, rpt)])

    return sk(msg_pad, dst_pad, zeros128)


def _node_prep_body(x_ref, w_ref, b_ref, o_ref):
    h = jnp.dot(x_ref[...], w_ref[...],
                preferred_element_type=jnp.float32) + b_ref[...]
    o_ref[...] = jnp.concatenate([h, jnp.zeros_like(h)], axis=1)


def _edge_prep_body(ea_ref, w_ref, b_ref, o_ref):
    h = jnp.dot(ea_ref[...], w_ref[...],
                preferred_element_type=jnp.float32) + b_ref[...]
    o_ref[...] = jnp.maximum(h, 0.0).T


def _wgen_body(het_ref, k2_ref, o_ref):
    he = het_ref[...].astype(jnp.bfloat16)
    w = jnp.dot(k2_ref[...], he, preferred_element_type=jnp.float32)
    o_ref[...] = w.astype(jnp.bfloat16)


def _msg_body(hs_ref, wt_ref, b2_ref, h_ref, root_ref, cb_ref, msg_ref, r_ref):
    hs = hs_ref[:, :WIDTH]
    hst = hs.T
    acc = jnp.zeros((WIDTH, EB), jnp.float32)
    for i in range(WIDTH):
        w = wt_ref[i * WIDTH:(i + 1) * WIDTH, :].astype(jnp.float32)
        acc = acc + w * hst[i:i + 1, :]
    msg = acc.T + jnp.dot(hs, b2_ref[...], preferred_element_type=jnp.float32)
    col = jax.lax.broadcasted_iota(jnp.int32, (EB, WIDTH), 1)
    ones_col = jnp.where(col == 0, 1.0, 0.0)
    msg_ref[...] = jnp.concatenate([msg, ones_col], axis=1)
    r_ref[...] = jnp.dot(h_ref[:, :WIDTH], root_ref[...],
                         preferred_element_type=jnp.float32) + cb_ref[...]


def _update1_body(a0_ref, a1_ref, r_ref, h_ref, inv_ref):
    c = a0_ref[:, WIDTH:WIDTH + 1] + a1_ref[:, WIDTH:WIDTH + 1]
    inv = 1.0 / jnp.maximum(c, 1.0)
    h = jnp.maximum(
        (a0_ref[:, :WIDTH] + a1_ref[:, :WIDTH]) * inv + r_ref[...], 0.0)
    h_ref[...] = jnp.concatenate([h, jnp.zeros_like(h)], axis=1)
    inv_ref[...] = inv


def _update_body(a0_ref, a1_ref, inv_ref, r_ref, o_ref):
    h = jnp.maximum(
        (a0_ref[:, :WIDTH] + a1_ref[:, :WIDTH]) * inv_ref[...] + r_ref[...],
        0.0)
    o_ref[...] = jnp.concatenate([h, jnp.zeros_like(h)], axis=1)


def _final_body(a0_ref, a1_ref, inv_ref, r_ref, w2_ref, b2_ref, o_ref):
    h = jnp.maximum(
        (a0_ref[:, :WIDTH] + a1_ref[:, :WIDTH]) * inv_ref[...] + r_ref[...],
        0.0)
    o_ref[...] = jnp.sum(h * w2_ref[...], axis=1, keepdims=True) + b2_ref[0, 0]


def _part_specs(bn):
    off = NPAD // bn
    return (pl.BlockSpec((bn, 2 * WIDTH), lambda i: (i, 0)),
            pl.BlockSpec((bn, 2 * WIDTH), lambda i: (i + off, 0)))


def kernel(x, edge_index, edge_attr, fc1_W, fc1_b, k1_W, k1_b, k2_W, k2_b,
           root, conv_b, fc2_W, fc2_b):
    N = x.shape[0]
    E = edge_attr.shape[0]
    src = jnp.pad(edge_index[0], (0, EPAD - E))
    dst = jnp.pad(edge_index[1], (0, EPAD - E), constant_values=NPAD - 1)
    ea_pad = jnp.pad(edge_attr, ((0, EPAD - E), (0, 0)))
    x_pad = jnp.pad(x, ((0, NPAD - N), (0, 0)))
    n_eblk = EPAD // EB
    nb = NPAD // n_eblk

    zeros128 = jnp.zeros((NPAD, 2 * WIDTH), jnp.float32)

    h = pl.pallas_call(
        _node_prep_body,
        grid=(NPAD // BN,),
        in_specs=[
            pl.BlockSpec((BN, 3), lambda i: (i, 0)),
            pl.BlockSpec((3, WIDTH), lambda i: (0, 0)),
            pl.BlockSpec((1, WIDTH), lambda i: (0, 0)),
        ],
        out_specs=pl.BlockSpec((BN, 2 * WIDTH), lambda i: (i, 0)),
        out_shape=jax.ShapeDtypeStruct((NPAD, 2 * WIDTH), jnp.float32),
    )(x_pad, fc1_W, fc1_b.reshape(1, WIDTH))

    het = pl.pallas_call(
        _edge_prep_body,
        grid=(n_eblk,),
        in_specs=[
            pl.BlockSpec((EB, 3), lambda i: (i, 0)),
            pl.BlockSpec((3, WIDTH), lambda i: (0, 0)),
            pl.BlockSpec((1, WIDTH), lambda i: (0, 0)),
        ],
        out_specs=pl.BlockSpec((WIDTH, EB), lambda i: (0, i)),
        out_shape=jax.ShapeDtypeStruct((WIDTH, EPAD), jnp.float32),
    )(ea_pad, k1_W, k1_b.reshape(1, WIDTH))

    k2r2 = k2_W.T.astype(jnp.bfloat16)
    wt = pl.pallas_call(
        _wgen_body,
        grid=(n_eblk,),
        in_specs=[
            pl.BlockSpec((WIDTH, EB), lambda i: (0, i)),
            pl.BlockSpec((WIDTH * WIDTH, WIDTH), lambda i: (0, 0)),
        ],
        out_specs=pl.BlockSpec((WIDTH * WIDTH, EB), lambda i: (0, i)),
        out_shape=jax.ShapeDtypeStruct((WIDTH * WIDTH, EPAD), jnp.bfloat16),
    )(het, k2r2)

    B2 = k2_b.reshape(WIDTH, WIDTH)

    msg_call = pl.pallas_call(
        _msg_body,
        grid=(n_eblk,),
        in_specs=[
            pl.BlockSpec((EB, 2 * WIDTH), lambda i: (i, 0)),
            pl.BlockSpec((WIDTH * WIDTH, EB), lambda i: (0, i)),
            pl.BlockSpec((WIDTH, WIDTH), lambda i: (0, 0)),
            pl.BlockSpec((nb, 2 * WIDTH), lambda i: (i, 0)),
            pl.BlockSpec((WIDTH, WIDTH), lambda i: (0, 0)),
            pl.BlockSpec((1, WIDTH), lambda i: (0, 0)),
        ],
        out_specs=[
            pl.BlockSpec((EB, 2 * WIDTH), lambda i: (i, 0)),
            pl.BlockSpec((nb, WIDTH), lambda i: (i, 0)),
        ],
        out_shape=[
            jax.ShapeDtypeStruct((EPAD, 2 * WIDTH), jnp.float32),
            jax.ShapeDtypeStruct((NPAD, WIDTH), jnp.float32),
        ],
    )

    cbr = conv_b.reshape(1, WIDTH)
    p0_spec, p1_spec = _part_specs(BN)
    coff = NPAD // BN
    inv_cnt = None
    for layer in range(DEPTH):
        hs = _sc_gather(h, src)
        msg, r = msg_call(hs, wt, B2, h, root, cbr)
        parts = _sc_scatter(msg, dst, zeros128)
        if layer == 0:
            h, inv_cnt = pl.pallas_call(
                _update1_body,
                grid=(NPAD // BN,),
                in_specs=[
                    p0_spec, p1_spec,
                    pl.BlockSpec((BN, WIDTH), lambda i: (i, 0)),
                ],
                out_specs=[
                    pl.BlockSpec((BN, 2 * WIDTH), lambda i: (i, 0)),
                    pl.BlockSpec((BN, 1), lambda i: (i, 0)),
                ],
                out_shape=[
                    jax.ShapeDtypeStruct((NPAD, 2 * WIDTH), jnp.float32),
                    jax.ShapeDtypeStruct((NPAD, 1), jnp.float32),
                ],
            )(parts, parts, r)
        elif layer < DEPTH - 1:
            h = pl.pallas_call(
                _update_body,
                grid=(NPAD // BN,),
                in_specs=[
                    p0_spec, p1_spec,
                    pl.BlockSpec((BN, 1), lambda i: (i, 0)),
                    pl.BlockSpec((BN, WIDTH), lambda i: (i, 0)),
                ],
                out_specs=pl.BlockSpec((BN, 2 * WIDTH), lambda i: (i, 0)),
                out_shape=jax.ShapeDtypeStruct((NPAD, 2 * WIDTH), jnp.float32),
            )(parts, parts, inv_cnt, r)
        else:
            out_pad = pl.pallas_call(
                _final_body,
                grid=(NPAD // BN,),
                in_specs=[
                    p0_spec, p1_spec,
                    pl.BlockSpec((BN, 1), lambda i: (i, 0)),
                    pl.BlockSpec((BN, WIDTH), lambda i: (i, 0)),
                    pl.BlockSpec((1, WIDTH), lambda i: (0, 0)),
                    pl.BlockSpec((1, 1), lambda i: (0, 0)),
                ],
                out_specs=pl.BlockSpec((BN, 1), lambda i: (i, 0)),
                out_shape=jax.ShapeDtypeStruct((NPAD, 1), jnp.float32),
            )(parts, parts, inv_cnt, r, fc2_W.reshape(1, WIDTH),
              fc2_b.reshape(1, 1))
            out = out_pad[:N]
    return out
